# baseline (speedup 1.0000x reference)
import functools

import jax
import jax.numpy as jnp
from jax import lax
from jax.experimental import pallas as pl
from jax.experimental.pallas import tpu as pltpu
from jax.experimental.pallas import tpu_sc as plsc

_L = 16
_DP = 128
_CW = 384
_SLOTS = 640
_BIG = 0x7FFFFFF


def _ceil_mult(x, m):
    return (x + m - 1) // m * m


@functools.lru_cache(maxsize=None)
def _build(B, D, NB, NBOOK, NAUTH):
    info = plsc.get_sparse_core_info()
    NC, NS = info.num_cores, info.num_subcores
    NW = NC * NS
    bpw = B // NW
    W = 3 * D + 1
    BD = B + _DP
    mesh = plsc.VectorSubcoreMesh(core_axis_name="c", subcore_axis_name="s")

    def geom(n):
        slab = _ceil_mult(_ceil_mult(n, NW) // NW, _DP)
        nch = (slab + _CW - 1) // _CW
        tail0 = (n // _DP) * _DP
        clamp = max(0, tail0 - _CW)
        return slab, nch, tail0, clamp

    BSLAB, BNCH, BTAIL0, BCLAMP = geom(NBOOK)
    ASLAB, ANCH, ATAIL0, ACLAMP = geom(NAUTH)

    @functools.partial(
        pl.kernel,
        mesh=mesh,
        compiler_params=pltpu.CompilerParams(needs_layout_passes=False),
        out_type=(
            jax.ShapeDtypeStruct((BD, _DP), jnp.float32),
            jax.ShapeDtypeStruct((BD, _DP), jnp.float32),
        ),
        scratch_types=[
            pltpu.VMEM((B,), jnp.int32),
            pltpu.VMEM((3 * 64, _DP), jnp.float32),
            pltpu.VMEM((_SLOTS, _DP), jnp.float32),
            pltpu.VMEM((_SLOTS,), jnp.int32),
            pltpu.VMEM((_SLOTS // _DP, _DP), jnp.int32),
            pltpu.VMEM((_DP,), jnp.int32),
            pltpu.VMEM((_DP,), jnp.int32),
            pltpu.SemaphoreType.DMA,
        ],
    )
    def gather_k(isbn_hbm, auth_hbm, bt_hbm, at_hbm, btail_hbm, atail_hbm,
                 bout_hbm, aout_hbm,
                 idx_v, chunk_v, rows_v, sr_v, pidx_v, ccc_v, ccs_v, sem):
        wid = lax.axis_index("s") * NC + lax.axis_index("c")
        lane = jnp.arange(_L, dtype=jnp.int32)

        def run_table(idx_hbm, tab_hbm, tail_hbm, out_hbm,
                      slab, nch, tail0, clamp):
            lo = wid * slab
            hi = lo + slab
            for k in range(_SLOTS // _L):
                sr_v[pl.ds(k * _L, _L)] = jnp.full((_L,), _BIG, jnp.int32)
            for k in range(_SLOTS // _DP):
                for q in range(_DP // _L):
                    pidx_v[k, pl.ds(q * _L, _L)] = jnp.full(
                        (_L,), B + 7, jnp.int32)
            pltpu.sync_copy(idx_hbm, idx_v)

            def pre_body(g, cnt):
                v = idx_v[pl.ds(g * _L, _L)]
                m = (v >= lo) & (v < hi)
                n = jnp.sum(jnp.where(m, 1, 0))

                @pl.when(n > 0)
                def _():
                    pos = cnt + plsc.cumsum(jnp.where(m, 1, 0)) - 1
                    pos = jnp.minimum(pos, _SLOTS - 1)
                    plsc.store_scatter(sr_v, [pos], v, mask=m)
                    plsc.store_scatter(
                        pidx_v, [pos >> 7, pos & 127], g * _L + lane, mask=m)

                return cnt + n

            cnt = lax.fori_loop(0, B // _L, pre_body, jnp.int32(0))
            ngrp = (cnt + _L - 1) // _L

            def extract_window(o, width):
                def cl_body(k, ccnt):
                    rv = sr_v[pl.ds(k * _L, _L)]
                    m = (rv >= o) & (rv < o + width)
                    n = jnp.sum(jnp.where(m, 1, 0))

                    @pl.when(n > 0)
                    def _():
                        pos = ccnt + plsc.cumsum(jnp.where(m, 1, 0)) - 1
                        pos = jnp.minimum(pos, _DP - 1)
                        plsc.store_scatter(ccc_v, [pos], rv - o, mask=m)
                        plsc.store_scatter(ccs_v, [pos], k * _L + lane, mask=m)

                    return ccnt + n

                ccnt = lax.fori_loop(0, _SLOTS // _L, cl_body, jnp.int32(0))
                ccnt = jnp.minimum(ccnt, _DP)

                def ex_body(e, _):
                    cc = ccc_v[pl.ds(e * _L, _L)]
                    slot = ccs_v[pl.ds(e * _L, _L)]
                    em = (e * _L + lane) < ccnt
                    sub = (cc >> 7) * 64
                    col = cc & 127
                    for c in range(D):
                        val = plsc.load_gather(
                            chunk_v, [sub + c, col], mask=em)
                        plsc.store_scatter(
                            rows_v, [slot, jnp.full((_L,), c, jnp.int32)],
                            val, mask=em)
                    return 0

                lax.fori_loop(0, (ccnt + _L - 1) // _L, ex_body, 0)

            def ch_body(j, _):
                o = jnp.minimum(lo + j * _CW, clamp)
                for i in range(_CW // _DP):
                    pltpu.sync_copy(
                        tab_hbm.at[:, pl.ds(o + i * _DP, _DP)],
                        chunk_v.at[pl.ds(i * 64, 64), :])
                extract_window(o, _CW)
                return 0

            lax.fori_loop(0, nch, ch_body, 0)

            pltpu.sync_copy(tail_hbm, chunk_v.at[pl.ds(0, 64), :])
            extract_window(jnp.int32(tail0), _DP)

            cps = []
            for k in range(_SLOTS // _DP):
                cps.append(pltpu.async_copy(
                    rows_v.at[pl.ds(k * _DP, _DP)],
                    out_hbm.at[pidx_v.at[k]],
                    sem))
            for cp in cps:
                cp.wait()

        run_table(isbn_hbm, bt_hbm, btail_hbm, bout_hbm,
                  BSLAB, BNCH, BTAIL0, BCLAMP)
        run_table(auth_hbm, at_hbm, atail_hbm, aout_hbm,
                  ASLAB, ANCH, ATAIL0, ACLAMP)

    @functools.partial(
        pl.kernel,
        mesh=mesh,
        compiler_params=pltpu.CompilerParams(needs_layout_passes=False),
        out_type=jax.ShapeDtypeStruct((B * W,), jnp.float32),
        scratch_types=[
            pltpu.VMEM((bpw,), jnp.float32),
            pltpu.VMEM((bpw // _DP, _DP), jnp.int32),
            pltpu.VMEM((NB + 4,), jnp.float32),
            pltpu.VMEM((2 * _L,), jnp.float32),
            pltpu.VMEM((_DP, _DP), jnp.float32),
            pltpu.VMEM((_DP, _DP), jnp.float32),
            pltpu.VMEM((_DP, _DP), jnp.float32),
            pltpu.VMEM((_DP * W,), jnp.float32),
            pltpu.SemaphoreType.DMA,
        ],
    )
    def asm_k(year_hbm, bout_hbm, aout_hbm, ytab_hbm, bpad_hbm, consts_hbm,
              out_hbm,
              year_v, ybkt_v, bpad_v, consts_v, bst_v, ast_v, yst_v, blk_v,
              sem):
        wid = lax.axis_index("s") * NC + lax.axis_index("c")
        lane = jnp.arange(_L, dtype=jnp.int32)
        base = wid * bpw
        pltpu.sync_copy(year_hbm.at[pl.ds(base, bpw)], year_v)
        pltpu.sync_copy(bpad_hbm, bpad_v)
        pltpu.sync_copy(consts_hbm, consts_v)

        mean = consts_v[pl.ds(0, _L)]
        std = consts_v[pl.ds(_L, _L)]
        scale = jnp.float32(NB - 1)
        for c in range(bpw // _L):
            y = year_v[pl.ds(c * _L, _L)]
            j = jnp.clip((y * scale).astype(jnp.int32) + 1, 0, NB)
            for _ in range(2):
                hi = plsc.load_gather(bpad_v, [j + 1])
                lo2 = plsc.load_gather(bpad_v, [j])
                j = j + jnp.where(hi <= y, 1, 0) - jnp.where(lo2 > y, 1, 0)
            ybkt_v[c // 8, pl.ds((c % 8) * _L, _L)] = j

        for g in range(bpw // _DP):
            r0 = base + g * _DP
            c0 = pltpu.async_copy(bout_hbm.at[pl.ds(r0, _DP)], bst_v, sem)
            c0.wait()
            c1 = pltpu.async_copy(aout_hbm.at[pl.ds(r0, _DP)], ast_v, sem)
            c1.wait()
            c2 = pltpu.async_copy(ytab_hbm.at[ybkt_v.at[g]], yst_v, sem)
            c2.wait()

            def row_body(r, _):
                dst = r * W
                for t, buf in ((0, bst_v), (1, ast_v), (2, yst_v)):
                    for k in range(D // _L):
                        v = buf[r, pl.ds(k * _L, _L)]
                        plsc.store_scatter(
                            blk_v, [dst + t * D + k * _L + lane], v)
                return 0

            lax.fori_loop(0, _DP, row_body, 0)
            for q in range(_DP // _L):
                y = year_v[pl.ds(g * _DP + q * _L, _L)]
                ny = (y - mean) / std
                plsc.store_scatter(
                    blk_v, [(q * _L + lane) * W + 3 * D], ny)
            pltpu.sync_copy(
                blk_v, out_hbm.at[pl.ds((r0) * W, _DP * W)])

    return gather_k, asm_k


def kernel(isbn_idx, author_idx, year_of_publication, book_table,
           author_table, year_table, boundaries, year_mean, year_std):
    B = isbn_idx.shape[0]
    D = book_table.shape[1]
    NB = boundaries.shape[0]
    NBOOK = book_table.shape[0]
    NAUTH = author_table.shape[0]
    gather_k, asm_k = _build(B, D, NB, NBOOK, NAUTH)

    def tail_of(tab):
        t0 = (tab.shape[0] // _DP) * _DP
        tl = tab[t0:]
        return jnp.pad(tl, ((0, _DP - tl.shape[0]), (0, 0))).T

    book_out, auth_out = gather_k(
        isbn_idx, author_idx, book_table.T, author_table.T,
        tail_of(book_table), tail_of(author_table))

    year128 = jnp.pad(year_table, ((0, 0), (0, _DP - D)))
    neg = jnp.full((1,), -jnp.inf, dtype=jnp.float32)
    pos = jnp.full((3,), jnp.inf, dtype=jnp.float32)
    bpad = jnp.concatenate([neg, boundaries.astype(jnp.float32), pos])
    consts = jnp.concatenate([
        jnp.full((_L,), year_mean, dtype=jnp.float32),
        jnp.full((_L,), year_std, dtype=jnp.float32),
    ])
    out = asm_k(year_of_publication, book_out, auth_out, year128,
                bpad, consts)
    return out.reshape(B, 3 * D + 1)


# speedup vs baseline: 1.2765x; 1.2765x over previous
import functools

import jax
import jax.numpy as jnp
from jax import lax
from jax.experimental import pallas as pl
from jax.experimental.pallas import tpu as pltpu
from jax.experimental.pallas import tpu_sc as plsc

_L = 16
_DP = 128
_CW = 256
_SLOTS = 640
_BIG = 0x7FFFFFF


def _ceil_mult(x, m):
    return (x + m - 1) // m * m


@functools.lru_cache(maxsize=None)
def _build(B, D, NB, NBOOK, NAUTH):
    info = plsc.get_sparse_core_info()
    NC, NS = info.num_cores, info.num_subcores
    NW = NC * NS
    bpw = B // NW
    W = 3 * D + 1
    BD = B + _DP
    mesh = plsc.VectorSubcoreMesh(core_axis_name="c", subcore_axis_name="s")

    def geom(n):
        slab = _ceil_mult(_ceil_mult(n, NW) // NW, _DP)
        nch = (slab + _CW - 1) // _CW
        tail0 = (n // _DP) * _DP
        clamp = max(0, tail0 - _CW)
        return slab, nch, tail0, clamp

    BSLAB, BNCH, BTAIL0, BCLAMP = geom(NBOOK)
    ASLAB, ANCH, ATAIL0, ACLAMP = geom(NAUTH)

    @functools.partial(
        pl.kernel,
        mesh=mesh,
        compiler_params=pltpu.CompilerParams(needs_layout_passes=False),
        out_type=(
            jax.ShapeDtypeStruct((BD, _DP), jnp.float32),
            jax.ShapeDtypeStruct((BD, _DP), jnp.float32),
        ),
        scratch_types=[
            pltpu.VMEM((2048,), jnp.int32),
            pltpu.VMEM((4 * 64, _DP), jnp.float32),
            pltpu.VMEM((_SLOTS, _DP), jnp.float32),
            pltpu.VMEM((_SLOTS,), jnp.int32),
            pltpu.VMEM((_SLOTS // _DP, _DP), jnp.int32),
            pltpu.VMEM((_DP,), jnp.int32),
            pltpu.VMEM((_DP,), jnp.int32),
            pltpu.SemaphoreType.DMA,
            pltpu.SemaphoreType.DMA,
        ],
    )
    def gather_k(isbn_hbm, auth_hbm, bt_hbm, at_hbm, btail_hbm, atail_hbm,
                 bout_hbm, aout_hbm,
                 idx_v, chunk_v, rows_v, sr_v, pidx_v, ccc_v, ccs_v,
                 sem0, sem1):
        wid = lax.axis_index("s") * NC + lax.axis_index("c")
        lane = jnp.arange(_L, dtype=jnp.int32)

        def run_table(idx_hbm, tab_hbm, tail_hbm, out_hbm,
                      slab, nch, tail0, clamp):
            lo = wid * slab
            hi = lo + slab
            for k in range(_SLOTS // _L):
                sr_v[pl.ds(k * _L, _L)] = jnp.full((_L,), _BIG, jnp.int32)
            for k in range(_SLOTS // _DP):
                for q in range(_DP // _L):
                    pidx_v[k, pl.ds(q * _L, _L)] = jnp.full(
                        (_L,), B + 7, jnp.int32)

            def pre_outer(p, cnt):
                pltpu.sync_copy(idx_hbm.at[pl.ds(p * 2048, 2048)], idx_v)

                def pre_body(g, cnt):
                    v = idx_v[pl.ds(g * _L, _L)]
                    m = (v >= lo) & (v < hi)
                    n = jnp.sum(jnp.where(m, 1, 0))

                    @pl.when(n > 0)
                    def _():
                        pos = cnt + plsc.cumsum(jnp.where(m, 1, 0)) - 1
                        pos = jnp.minimum(pos, _SLOTS - 1)
                        plsc.store_scatter(sr_v, [pos], v, mask=m)
                        plsc.store_scatter(
                            pidx_v, [pos >> 7, pos & 127],
                            p * 2048 + g * _L + lane, mask=m)

                    return cnt + n

                return lax.fori_loop(0, 2048 // _L, pre_body, cnt)

            lax.fori_loop(0, B // 2048, pre_outer, jnp.int32(0))

            def extract_chunk(o, pair, width):
                def cl_body(k, ccnt):
                    rv = sr_v[pl.ds(k * _L, _L)]
                    m = (rv >= o) & (rv < o + width)
                    n = jnp.sum(jnp.where(m, 1, 0))

                    @pl.when(n > 0)
                    def _():
                        pos = ccnt + plsc.cumsum(jnp.where(m, 1, 0)) - 1
                        pos = jnp.minimum(pos, _DP - 1)
                        plsc.store_scatter(ccc_v, [pos], rv - o, mask=m)
                        plsc.store_scatter(ccs_v, [pos], k * _L + lane, mask=m)

                    return ccnt + n

                ccnt = lax.fori_loop(0, _SLOTS // _L, cl_body, jnp.int32(0))
                ccnt = jnp.minimum(ccnt, _DP)

                def ex_body(e, _):
                    cc = ccc_v[pl.ds(e * _L, _L)]
                    slot = ccs_v[pl.ds(e * _L, _L)]
                    em = (e * _L + lane) < ccnt
                    sub = ((pair * 2 + (cc >> 7)) & 3) * 64
                    col = cc & 127
                    for c in range(D):
                        val = plsc.load_gather(
                            chunk_v, [sub + c, col], mask=em)
                        plsc.store_scatter(
                            rows_v, [slot, jnp.full((_L,), c, jnp.int32)],
                            val, mask=em)
                    return 0

                lax.fori_loop(0, (ccnt + _L - 1) // _L, ex_body, 0)

            def off(j):
                return jnp.minimum(lo + j * _CW, clamp)

            def fire_dyn(j):
                def issue(sem):
                    o = off(j)
                    par = j & 1
                    for i in range(2):
                        dst0 = (par * 2 + i) * 64
                        pltpu.async_copy(
                            tab_hbm.at[:, pl.ds(o + i * _DP, _DP)],
                            chunk_v.at[pl.ds(dst0, 64), :], sem)

                @pl.when((j & 1) == 0)
                def _():
                    issue(sem0)

                @pl.when((j & 1) == 1)
                def _():
                    issue(sem1)

            def drain_dyn(j):
                def dr(sem):
                    o = off(j)
                    par = j & 1
                    for i in range(2):
                        dst0 = (par * 2 + i) * 64
                        pltpu.make_async_copy(
                            tab_hbm.at[:, pl.ds(o + i * _DP, _DP)],
                            chunk_v.at[pl.ds(dst0, 64), :], sem).wait()

                @pl.when((j & 1) == 0)
                def _():
                    dr(sem0)

                @pl.when((j & 1) == 1)
                def _():
                    dr(sem1)

            fire_dyn(jnp.int32(0))

            def ch_body(j, _):
                @pl.when(j + 1 < nch)
                def _():
                    fire_dyn(j + 1)

                drain_dyn(j)
                extract_chunk(off(j), j & 1, _CW)
                return 0

            lax.fori_loop(0, nch, ch_body, 0)

            pltpu.sync_copy(tail_hbm, chunk_v.at[pl.ds(0, 64), :])
            extract_chunk(jnp.int32(tail0), jnp.int32(0), _DP)

            cps = []
            for k in range(_SLOTS // _DP):
                cps.append(pltpu.async_copy(
                    rows_v.at[pl.ds(k * _DP, _DP)],
                    out_hbm.at[pidx_v.at[k]],
                    sem0))
            for cp in cps:
                cp.wait()

        run_table(isbn_hbm, bt_hbm, btail_hbm, bout_hbm,
                  BSLAB, BNCH, BTAIL0, BCLAMP)
        run_table(auth_hbm, at_hbm, atail_hbm, aout_hbm,
                  ASLAB, ANCH, ATAIL0, ACLAMP)

    @functools.partial(
        pl.kernel,
        mesh=mesh,
        compiler_params=pltpu.CompilerParams(needs_layout_passes=False),
        out_type=jax.ShapeDtypeStruct((B * W,), jnp.float32),
        scratch_types=[
            pltpu.VMEM((bpw,), jnp.float32),
            pltpu.VMEM((bpw // _DP, _DP), jnp.int32),
            pltpu.VMEM((NB + 4,), jnp.float32),
            pltpu.VMEM((2 * _L,), jnp.float32),
            pltpu.VMEM((_DP, _DP), jnp.float32),
            pltpu.VMEM((_DP, _DP), jnp.float32),
            pltpu.VMEM((_DP, _DP), jnp.float32),
            pltpu.VMEM((_DP * W,), jnp.float32),
            pltpu.SemaphoreType.DMA,
            pltpu.SemaphoreType.DMA,
            pltpu.SemaphoreType.DMA,
        ],
    )
    def asm_k(year_hbm, bout_hbm, aout_hbm, ytab_hbm, bpad_hbm, consts_hbm,
              out_hbm,
              year_v, ybkt_v, bpad_v, consts_v, bst_v, ast_v, yst_v, blk_v,
              sem, sem1, sem2):
        wid = lax.axis_index("s") * NC + lax.axis_index("c")
        lane = jnp.arange(_L, dtype=jnp.int32)
        base = wid * bpw
        pltpu.sync_copy(year_hbm.at[pl.ds(base, bpw)], year_v)
        pltpu.sync_copy(bpad_hbm, bpad_v)
        pltpu.sync_copy(consts_hbm, consts_v)

        mean = consts_v[pl.ds(0, _L)]
        std = consts_v[pl.ds(_L, _L)]
        scale = jnp.float32(NB - 1)
        for c in range(bpw // _L):
            y = year_v[pl.ds(c * _L, _L)]
            j = jnp.clip((y * scale).astype(jnp.int32) + 1, 0, NB)
            for _ in range(2):
                hi = plsc.load_gather(bpad_v, [j + 1])
                lo2 = plsc.load_gather(bpad_v, [j])
                j = j + jnp.where(hi <= y, 1, 0) - jnp.where(lo2 > y, 1, 0)
            ybkt_v[c // 8, pl.ds((c % 8) * _L, _L)] = j

        for g in range(bpw // _DP):
            r0 = base + g * _DP
            c0 = pltpu.async_copy(bout_hbm.at[pl.ds(r0, _DP)], bst_v, sem)
            c1 = pltpu.async_copy(aout_hbm.at[pl.ds(r0, _DP)], ast_v, sem1)
            c2 = pltpu.async_copy(ytab_hbm.at[ybkt_v.at[g]], yst_v, sem2)
            c0.wait()
            c1.wait()
            c2.wait()

            def row_body(r, _):
                dst = r * W
                for t, buf in ((0, bst_v), (1, ast_v), (2, yst_v)):
                    for k in range(D // _L):
                        v = buf[r, pl.ds(k * _L, _L)]
                        plsc.store_scatter(
                            blk_v, [dst + t * D + k * _L + lane], v)
                return 0

            lax.fori_loop(0, _DP, row_body, 0)
            for q in range(_DP // _L):
                y = year_v[pl.ds(g * _DP + q * _L, _L)]
                ny = (y - mean) / std
                plsc.store_scatter(
                    blk_v, [(q * _L + lane) * W + 3 * D], ny)
            pltpu.sync_copy(
                blk_v, out_hbm.at[pl.ds((r0) * W, _DP * W)])

    return gather_k, asm_k


def kernel(isbn_idx, author_idx, year_of_publication, book_table,
           author_table, year_table, boundaries, year_mean, year_std):
    B = isbn_idx.shape[0]
    D = book_table.shape[1]
    NB = boundaries.shape[0]
    NBOOK = book_table.shape[0]
    NAUTH = author_table.shape[0]
    gather_k, asm_k = _build(B, D, NB, NBOOK, NAUTH)

    def tail_of(tab):
        t0 = (tab.shape[0] // _DP) * _DP
        tl = tab[t0:]
        return jnp.pad(tl, ((0, _DP - tl.shape[0]), (0, 0))).T

    book_out, auth_out = gather_k(
        isbn_idx, author_idx, book_table.T, author_table.T,
        tail_of(book_table), tail_of(author_table))

    year128 = jnp.pad(year_table, ((0, 0), (0, _DP - D)))
    neg = jnp.full((1,), -jnp.inf, dtype=jnp.float32)
    pos = jnp.full((3,), jnp.inf, dtype=jnp.float32)
    bpad = jnp.concatenate([neg, boundaries.astype(jnp.float32), pos])
    consts = jnp.concatenate([
        jnp.full((_L,), year_mean, dtype=jnp.float32),
        jnp.full((_L,), year_std, dtype=jnp.float32),
    ])
    out = asm_k(year_of_publication, book_out, auth_out, year128,
                bpad, consts)
    return out.reshape(B, 3 * D + 1)


# speedup vs baseline: 1.2773x; 1.0006x over previous
import functools

import jax
import jax.numpy as jnp
from jax import lax
from jax.experimental import pallas as pl
from jax.experimental.pallas import tpu as pltpu
from jax.experimental.pallas import tpu_sc as plsc

_L = 16
_DP = 128
_CW = 256
_SLOTS = 640
_BIG = 0x7FFFFFF


def _ceil_mult(x, m):
    return (x + m - 1) // m * m


@functools.lru_cache(maxsize=None)
def _build(B, D, NB, NBOOK, NAUTH):
    info = plsc.get_sparse_core_info()
    NC, NS = info.num_cores, info.num_subcores
    NW = NC * NS
    bpw = B // NW
    W = 3 * D + 1
    BD = B + _DP
    mesh = plsc.VectorSubcoreMesh(core_axis_name="c", subcore_axis_name="s")

    def geom(n):
        slab = _ceil_mult(_ceil_mult(n, NW) // NW, _DP)
        nch = (slab + _CW - 1) // _CW
        tail0 = (n // _DP) * _DP
        clamp = max(0, tail0 - _CW)
        return slab, nch, tail0, clamp

    BSLAB, BNCH, BTAIL0, BCLAMP = geom(NBOOK)
    ASLAB, ANCH, ATAIL0, ACLAMP = geom(NAUTH)

    @functools.partial(
        pl.kernel,
        mesh=mesh,
        compiler_params=pltpu.CompilerParams(needs_layout_passes=False),
        out_type=(
            jax.ShapeDtypeStruct((BD, _DP), jnp.float32),
            jax.ShapeDtypeStruct((BD, _DP), jnp.float32),
        ),
        scratch_types=[
            pltpu.VMEM((2048,), jnp.int32),
            pltpu.VMEM((4 * 64, _DP), jnp.float32),
            pltpu.VMEM((_SLOTS, _DP), jnp.float32),
            pltpu.VMEM((_SLOTS,), jnp.int32),
            pltpu.VMEM((_SLOTS // _DP, _DP), jnp.int32),
            pltpu.VMEM((_DP,), jnp.int32),
            pltpu.VMEM((_DP,), jnp.int32),
            pltpu.SemaphoreType.DMA,
            pltpu.SemaphoreType.DMA,
        ],
    )
    def gather_k(isbn_hbm, auth_hbm, bt_hbm, at_hbm, btail_hbm, atail_hbm,
                 bout_hbm, aout_hbm,
                 idx_v, chunk_v, rows_v, sr_v, pidx_v, ccc_v, ccs_v,
                 sem0, sem1):
        wid = lax.axis_index("s") * NC + lax.axis_index("c")
        lane = jnp.arange(_L, dtype=jnp.int32)

        def run_table(idx_hbm, tab_hbm, tail_hbm, out_hbm,
                      slab, nch, tail0, clamp):
            lo = wid * slab
            hi = lo + slab
            for k in range(_SLOTS // _L):
                sr_v[pl.ds(k * _L, _L)] = jnp.full((_L,), _BIG, jnp.int32)
            for k in range(_SLOTS // _DP):
                for q in range(_DP // _L):
                    pidx_v[k, pl.ds(q * _L, _L)] = jnp.full(
                        (_L,), B + 7, jnp.int32)

            def pre_outer(p, cnt):
                pltpu.sync_copy(idx_hbm.at[pl.ds(p * 2048, 2048)], idx_v)

                def pre_body(g, cnt):
                    v = idx_v[pl.ds(g * _L, _L)]
                    m = (v >= lo) & (v < hi)
                    n = jnp.sum(jnp.where(m, 1, 0))

                    @pl.when(n > 0)
                    def _():
                        pos = cnt + plsc.cumsum(jnp.where(m, 1, 0)) - 1
                        pos = jnp.minimum(pos, _SLOTS - 1)
                        plsc.store_scatter(sr_v, [pos], v, mask=m)
                        plsc.store_scatter(
                            pidx_v, [pos >> 7, pos & 127],
                            p * 2048 + g * _L + lane, mask=m)

                    return cnt + n

                return lax.fori_loop(0, 2048 // _L, pre_body, cnt)

            lax.fori_loop(0, B // 2048, pre_outer, jnp.int32(0))

            def extract_chunk(o, pair, width):
                def cl_body(k, ccnt):
                    rv = sr_v[pl.ds(k * _L, _L)]
                    m = (rv >= o) & (rv < o + width)
                    n = jnp.sum(jnp.where(m, 1, 0))

                    @pl.when(n > 0)
                    def _():
                        pos = ccnt + plsc.cumsum(jnp.where(m, 1, 0)) - 1
                        pos = jnp.minimum(pos, _DP - 1)
                        plsc.store_scatter(ccc_v, [pos], rv - o, mask=m)
                        plsc.store_scatter(ccs_v, [pos], k * _L + lane, mask=m)

                    return ccnt + n

                ccnt = lax.fori_loop(0, _SLOTS // _L, cl_body, jnp.int32(0))
                ccnt = jnp.minimum(ccnt, _DP)

                def ex_body(e, _):
                    cc = ccc_v[pl.ds(e * _L, _L)]
                    slot = ccs_v[pl.ds(e * _L, _L)]
                    em = (e * _L + lane) < ccnt
                    sub = ((pair * 2 + (cc >> 7)) & 3) * 64
                    col = cc & 127
                    for c in range(D):
                        val = plsc.load_gather(
                            chunk_v, [sub + c, col], mask=em)
                        plsc.store_scatter(
                            rows_v, [slot, jnp.full((_L,), c, jnp.int32)],
                            val, mask=em)
                    return 0

                lax.fori_loop(0, (ccnt + _L - 1) // _L, ex_body, 0)

            def off(j):
                return jnp.minimum(lo + j * _CW, clamp)

            def fire_dyn(j):
                def issue(sem):
                    o = off(j)
                    par = j & 1
                    for i in range(2):
                        dst0 = (par * 2 + i) * 64
                        pltpu.async_copy(
                            tab_hbm.at[:, pl.ds(o + i * _DP, _DP)],
                            chunk_v.at[pl.ds(dst0, 64), :], sem)

                @pl.when((j & 1) == 0)
                def _():
                    issue(sem0)

                @pl.when((j & 1) == 1)
                def _():
                    issue(sem1)

            def drain_dyn(j):
                def dr(sem):
                    o = off(j)
                    par = j & 1
                    for i in range(2):
                        dst0 = (par * 2 + i) * 64
                        pltpu.make_async_copy(
                            tab_hbm.at[:, pl.ds(o + i * _DP, _DP)],
                            chunk_v.at[pl.ds(dst0, 64), :], sem).wait()

                @pl.when((j & 1) == 0)
                def _():
                    dr(sem0)

                @pl.when((j & 1) == 1)
                def _():
                    dr(sem1)

            fire_dyn(jnp.int32(0))

            def ch_body(j, _):
                @pl.when(j + 1 < nch)
                def _():
                    fire_dyn(j + 1)

                drain_dyn(j)
                extract_chunk(off(j), j & 1, _CW)
                return 0

            lax.fori_loop(0, nch, ch_body, 0)

            pltpu.sync_copy(tail_hbm, chunk_v.at[pl.ds(0, 64), :])
            extract_chunk(jnp.int32(tail0), jnp.int32(0), _DP)

            cps = []
            for k in range(_SLOTS // _DP):
                cps.append(pltpu.async_copy(
                    rows_v.at[pl.ds(k * _DP, _DP)],
                    out_hbm.at[pidx_v.at[k]],
                    sem0))
            for cp in cps:
                cp.wait()

        run_table(isbn_hbm, bt_hbm, btail_hbm, bout_hbm,
                  BSLAB, BNCH, BTAIL0, BCLAMP)
        run_table(auth_hbm, at_hbm, atail_hbm, aout_hbm,
                  ASLAB, ANCH, ATAIL0, ACLAMP)

    @functools.partial(
        pl.kernel,
        mesh=mesh,
        compiler_params=pltpu.CompilerParams(needs_layout_passes=False),
        out_type=jax.ShapeDtypeStruct((W, B), jnp.float32),
        scratch_types=[
            pltpu.VMEM((bpw,), jnp.float32),
            pltpu.VMEM((bpw // _DP, _DP), jnp.int32),
            pltpu.VMEM((NB + 4,), jnp.float32),
            pltpu.VMEM((2 * _L,), jnp.float32),
            pltpu.VMEM((_DP, _DP), jnp.float32),
            pltpu.VMEM((_DP, _DP), jnp.float32),
            pltpu.VMEM((_DP, _DP), jnp.float32),
            pltpu.VMEM((W, _DP), jnp.float32),
            pltpu.SemaphoreType.DMA,
            pltpu.SemaphoreType.DMA,
            pltpu.SemaphoreType.DMA,
        ],
    )
    def asm_k(year_hbm, bout_hbm, aout_hbm, ytab_hbm, bpad_hbm, consts_hbm,
              out_hbm,
              year_v, ybkt_v, bpad_v, consts_v, bst_v, ast_v, yst_v, blk_v,
              sem, sem1, sem2):
        wid = lax.axis_index("s") * NC + lax.axis_index("c")
        lane = jnp.arange(_L, dtype=jnp.int32)
        base = wid * bpw
        pltpu.sync_copy(year_hbm.at[pl.ds(base, bpw)], year_v)
        pltpu.sync_copy(bpad_hbm, bpad_v)
        pltpu.sync_copy(consts_hbm, consts_v)

        mean = consts_v[pl.ds(0, _L)]
        std = consts_v[pl.ds(_L, _L)]
        scale = jnp.float32(NB - 1)
        for c in range(bpw // _L):
            y = year_v[pl.ds(c * _L, _L)]
            j = jnp.clip((y * scale).astype(jnp.int32) + 1, 0, NB)
            for _ in range(2):
                hi = plsc.load_gather(bpad_v, [j + 1])
                lo2 = plsc.load_gather(bpad_v, [j])
                j = j + jnp.where(hi <= y, 1, 0) - jnp.where(lo2 > y, 1, 0)
            ybkt_v[c // 8, pl.ds((c % 8) * _L, _L)] = j

        for g in range(bpw // _DP):
            r0 = base + g * _DP
            c0 = pltpu.async_copy(bout_hbm.at[pl.ds(r0, _DP)], bst_v, sem)
            c1 = pltpu.async_copy(aout_hbm.at[pl.ds(r0, _DP)], ast_v, sem1)
            c2 = pltpu.async_copy(ytab_hbm.at[ybkt_v.at[g]], yst_v, sem2)
            c0.wait()
            c1.wait()
            c2.wait()

            def row_body(r, _):
                rv = jnp.full((_L,), 0, jnp.int32) + r
                for t, buf in ((0, bst_v), (1, ast_v), (2, yst_v)):
                    for k in range(D // _L):
                        v = buf[r, pl.ds(k * _L, _L)]
                        plsc.store_scatter(
                            blk_v, [t * D + k * _L + lane, rv], v)
                return 0

            lax.fori_loop(0, _DP, row_body, 0)
            for q in range(_DP // _L):
                y = year_v[pl.ds(g * _DP + q * _L, _L)]
                ny = (y - mean) / std
                plsc.store_scatter(
                    blk_v,
                    [jnp.full((_L,), 3 * D, jnp.int32), q * _L + lane], ny)
            pltpu.sync_copy(blk_v, out_hbm.at[:, pl.ds(r0, _DP)])

    return gather_k, asm_k


def kernel(isbn_idx, author_idx, year_of_publication, book_table,
           author_table, year_table, boundaries, year_mean, year_std):
    B = isbn_idx.shape[0]
    D = book_table.shape[1]
    NB = boundaries.shape[0]
    NBOOK = book_table.shape[0]
    NAUTH = author_table.shape[0]
    gather_k, asm_k = _build(B, D, NB, NBOOK, NAUTH)

    def tail_of(tab):
        t0 = (tab.shape[0] // _DP) * _DP
        tl = tab[t0:]
        return jnp.pad(tl, ((0, _DP - tl.shape[0]), (0, 0))).T

    book_out, auth_out = gather_k(
        isbn_idx, author_idx, book_table.T, author_table.T,
        tail_of(book_table), tail_of(author_table))

    year128 = jnp.pad(year_table, ((0, 0), (0, _DP - D)))
    neg = jnp.full((1,), -jnp.inf, dtype=jnp.float32)
    pos = jnp.full((3,), jnp.inf, dtype=jnp.float32)
    bpad = jnp.concatenate([neg, boundaries.astype(jnp.float32), pos])
    consts = jnp.concatenate([
        jnp.full((_L,), year_mean, dtype=jnp.float32),
        jnp.full((_L,), year_std, dtype=jnp.float32),
    ])
    out_t = asm_k(year_of_publication, book_out, auth_out, year128,
                  bpad, consts)
    return out_t.T


# speedup vs baseline: 1.3069x; 1.0232x over previous
import functools

import jax
import jax.numpy as jnp
from jax import lax
from jax.experimental import pallas as pl
from jax.experimental.pallas import tpu as pltpu
from jax.experimental.pallas import tpu_sc as plsc

_L = 16
_DP = 128
_CW = 256
_SLOTS = 640
_SUP = 384
_BIG = 0x7FFFFFF


def _ceil_mult(x, m):
    return (x + m - 1) // m * m


@functools.lru_cache(maxsize=None)
def _build(B, D, NB, NBOOK, NAUTH):
    info = plsc.get_sparse_core_info()
    NC, NS = info.num_cores, info.num_subcores
    NW = NC * NS
    bpw = B // NW
    W = 3 * D + 1
    BD = B + _DP
    mesh = plsc.VectorSubcoreMesh(core_axis_name="c", subcore_axis_name="s")

    def geom(n):
        slab = _ceil_mult(_ceil_mult(n, NW) // NW, _DP)
        nsup = (slab + 8 * _CW - 1) // (8 * _CW)
        tail0 = (n // _DP) * _DP
        clamp = max(0, tail0 - _CW)
        return slab, nsup, tail0, clamp

    BSLAB, BNSUP, BTAIL0, BCLAMP = geom(NBOOK)
    ASLAB, ANSUP, ATAIL0, ACLAMP = geom(NAUTH)

    @functools.partial(
        pl.kernel,
        mesh=mesh,
        compiler_params=pltpu.CompilerParams(needs_layout_passes=False),
        out_type=(
            jax.ShapeDtypeStruct((BD, _DP), jnp.float32),
            jax.ShapeDtypeStruct((BD, _DP), jnp.float32),
        ),
        scratch_types=[
            pltpu.VMEM((2048,), jnp.int32),
            pltpu.VMEM((4 * 64, _DP), jnp.float32),
            pltpu.VMEM((_SLOTS, _DP), jnp.float32),
            pltpu.VMEM((_SLOTS,), jnp.int32),
            pltpu.VMEM((_SLOTS // _DP, _DP), jnp.int32),
            pltpu.VMEM((_DP,), jnp.int32),
            pltpu.VMEM((_DP,), jnp.int32),
            pltpu.VMEM((_SUP,), jnp.int32),
            pltpu.VMEM((_SUP,), jnp.int32),
            pltpu.SemaphoreType.DMA,
            pltpu.SemaphoreType.DMA,
        ],
    )
    def gather_k(isbn_hbm, auth_hbm, bt_hbm, at_hbm, btail_hbm, atail_hbm,
                 bout_hbm, aout_hbm,
                 idx_v, chunk_v, rows_v, sr_v, pidx_v, ccc_v, ccs_v,
                 sr2_v, ss2_v, sem0, sem1):
        wid = lax.axis_index("s") * NC + lax.axis_index("c")
        lane = jnp.arange(_L, dtype=jnp.int32)

        def run_table(idx_hbm, tab_hbm, tail_hbm, out_hbm,
                      slab, nsup, tail0, clamp):
            nch = nsup * 8
            lo = wid * slab
            hi = lo + slab
            for k in range(_SLOTS // _L):
                sr_v[pl.ds(k * _L, _L)] = jnp.full((_L,), _BIG, jnp.int32)
            for k in range(_SLOTS // _DP):
                for q in range(_DP // _L):
                    pidx_v[k, pl.ds(q * _L, _L)] = jnp.full(
                        (_L,), B + 7, jnp.int32)

            def pre_outer(p, cnt):
                pltpu.sync_copy(idx_hbm.at[pl.ds(p * 2048, 2048)], idx_v)

                def pre_body(g, cnt):
                    v = idx_v[pl.ds(g * _L, _L)]
                    m = (v >= lo) & (v < hi)
                    n = jnp.sum(jnp.where(m, 1, 0))

                    @pl.when(n > 0)
                    def _():
                        pos = cnt + plsc.cumsum(jnp.where(m, 1, 0)) - 1
                        pos = jnp.minimum(pos, _SLOTS - 1)
                        plsc.store_scatter(sr_v, [pos], v, mask=m)
                        plsc.store_scatter(
                            pidx_v, [pos >> 7, pos & 127],
                            p * 2048 + g * _L + lane, mask=m)

                    return cnt + n

                return lax.fori_loop(0, 2048 // _L, pre_body, cnt)

            cnt = lax.fori_loop(0, B // 2048, pre_outer, jnp.int32(0))
            ngrp = jnp.minimum((cnt + _L - 1) // _L, _SLOTS // _L)

            def extract_chunk(o, pair, width, bound, from_super):
                def cl_body(k, ccnt):
                    for u in range(2):
                        g = k * 2 + u
                        if from_super:
                            rv = sr2_v[pl.ds(g * _L, _L)]
                            sv = ss2_v[pl.ds(g * _L, _L)]
                        else:
                            rv = sr_v[pl.ds(g * _L, _L)]
                            sv = g * _L + lane
                        m = (rv >= o) & (rv < o + width)
                        n = jnp.sum(jnp.where(m, 1, 0))

                        @pl.when(n > 0)
                        def _(ccnt=ccnt, m=m, rv=rv, sv=sv):
                            pos = ccnt + plsc.cumsum(jnp.where(m, 1, 0)) - 1
                            pos = jnp.minimum(pos, _DP - 1)
                            plsc.store_scatter(ccc_v, [pos], rv - o, mask=m)
                            plsc.store_scatter(ccs_v, [pos], sv, mask=m)

                        ccnt = ccnt + n
                    return ccnt

                ccnt = lax.fori_loop(0, (bound + 1) // 2, cl_body, jnp.int32(0))
                ccnt = jnp.minimum(ccnt, _DP)

                def ex_body(e, _):
                    cc = ccc_v[pl.ds(e * _L, _L)]
                    slot = ccs_v[pl.ds(e * _L, _L)]
                    em = (e * _L + lane) < ccnt
                    sub = ((pair * 2 + (cc >> 7)) & 3) * 64
                    col = cc & 127
                    for c in range(D):
                        val = plsc.load_gather(
                            chunk_v, [sub + c, col], mask=em)
                        plsc.store_scatter(
                            rows_v, [slot, jnp.full((_L,), c, jnp.int32)],
                            val, mask=em)
                    return 0

                lax.fori_loop(0, (ccnt + _L - 1) // _L, ex_body, 0)

            def off(j):
                return jnp.minimum(lo + j * _CW, clamp)

            def fire_dyn(j):
                def issue(sem):
                    o = off(j)
                    par = j & 1
                    for i in range(2):
                        dst0 = (par * 2 + i) * 64
                        pltpu.async_copy(
                            tab_hbm.at[:, pl.ds(o + i * _DP, _DP)],
                            chunk_v.at[pl.ds(dst0, 64), :], sem)

                @pl.when((j & 1) == 0)
                def _():
                    issue(sem0)

                @pl.when((j & 1) == 1)
                def _():
                    issue(sem1)

            def drain_dyn(j):
                def dr(sem):
                    o = off(j)
                    par = j & 1
                    for i in range(2):
                        dst0 = (par * 2 + i) * 64
                        pltpu.make_async_copy(
                            tab_hbm.at[:, pl.ds(o + i * _DP, _DP)],
                            chunk_v.at[pl.ds(dst0, 64), :], sem).wait()

                @pl.when((j & 1) == 0)
                def _():
                    dr(sem0)

                @pl.when((j & 1) == 1)
                def _():
                    dr(sem1)

            fire_dyn(jnp.int32(0))

            def sup_body(s, _):
                j0 = s * 8
                lo_s = off(j0)
                hi_s = off(j0 + 7) + _CW
                for k in range(_SUP // _L):
                    sr2_v[pl.ds(k * _L, _L)] = jnp.full(
                        (_L,), _BIG, jnp.int32)

                def sl_body(k, scnt):
                    for u in range(2):
                        g = k * 2 + u
                        rv = sr_v[pl.ds(g * _L, _L)]
                        m = (rv >= lo_s) & (rv < hi_s)
                        n = jnp.sum(jnp.where(m, 1, 0))

                        @pl.when(n > 0)
                        def _(scnt=scnt, m=m, rv=rv, g=g):
                            pos = scnt + plsc.cumsum(jnp.where(m, 1, 0)) - 1
                            pos = jnp.minimum(pos, _SUP - 1)
                            plsc.store_scatter(sr2_v, [pos], rv, mask=m)
                            plsc.store_scatter(
                                ss2_v, [pos], g * _L + lane, mask=m)

                        scnt = scnt + n
                    return scnt

                scnt = lax.fori_loop(0, (ngrp + 1) // 2, sl_body,
                                     jnp.int32(0))
                sgrp = jnp.minimum((scnt + _L - 1) // _L, _SUP // _L)

                def ch_body(j2, _):
                    j = j0 + j2

                    @pl.when(j + 1 < nch)
                    def _():
                        fire_dyn(j + 1)

                    drain_dyn(j)
                    extract_chunk(off(j), j & 1, _CW, sgrp, True)
                    return 0

                lax.fori_loop(0, 8, ch_body, 0)
                return 0

            lax.fori_loop(0, nsup, sup_body, 0)

            pltpu.sync_copy(tail_hbm, chunk_v.at[pl.ds(0, 64), :])
            extract_chunk(jnp.int32(tail0), jnp.int32(0), _DP, ngrp, False)

            cps = []
            for k in range(_SLOTS // _DP):
                cps.append(pltpu.async_copy(
                    rows_v.at[pl.ds(k * _DP, _DP)],
                    out_hbm.at[pidx_v.at[k]],
                    sem0))
            for cp in cps:
                cp.wait()

        run_table(isbn_hbm, bt_hbm, btail_hbm, bout_hbm,
                  BSLAB, BNSUP, BTAIL0, BCLAMP)
        run_table(auth_hbm, at_hbm, atail_hbm, aout_hbm,
                  ASLAB, ANSUP, ATAIL0, ACLAMP)

    @functools.partial(
        pl.kernel,
        mesh=mesh,
        compiler_params=pltpu.CompilerParams(needs_layout_passes=False),
        out_type=jax.ShapeDtypeStruct((W, B), jnp.float32),
        scratch_types=[
            pltpu.VMEM((bpw,), jnp.float32),
            pltpu.VMEM((bpw // _DP, _DP), jnp.int32),
            pltpu.VMEM((NB + 4,), jnp.float32),
            pltpu.VMEM((2 * _L,), jnp.float32),
            pltpu.VMEM((_DP, _DP), jnp.float32),
            pltpu.VMEM((_DP, _DP), jnp.float32),
            pltpu.VMEM((_DP, _DP), jnp.float32),
            pltpu.VMEM((W, _DP), jnp.float32),
            pltpu.SemaphoreType.DMA,
            pltpu.SemaphoreType.DMA,
            pltpu.SemaphoreType.DMA,
        ],
    )
    def asm_k(year_hbm, bout_hbm, aout_hbm, ytab_hbm, bpad_hbm, consts_hbm,
              out_hbm,
              year_v, ybkt_v, bpad_v, consts_v, bst_v, ast_v, yst_v, blk_v,
              sem, sem1, sem2):
        wid = lax.axis_index("s") * NC + lax.axis_index("c")
        lane = jnp.arange(_L, dtype=jnp.int32)
        base = wid * bpw
        pltpu.sync_copy(year_hbm.at[pl.ds(base, bpw)], year_v)
        pltpu.sync_copy(bpad_hbm, bpad_v)
        pltpu.sync_copy(consts_hbm, consts_v)

        mean = consts_v[pl.ds(0, _L)]
        std = consts_v[pl.ds(_L, _L)]
        scale = jnp.float32(NB - 1)
        for c in range(bpw // _L):
            y = year_v[pl.ds(c * _L, _L)]
            j = jnp.clip((y * scale).astype(jnp.int32) + 1, 0, NB)
            for _ in range(2):
                hi = plsc.load_gather(bpad_v, [j + 1])
                lo2 = plsc.load_gather(bpad_v, [j])
                j = j + jnp.where(hi <= y, 1, 0) - jnp.where(lo2 > y, 1, 0)
            ybkt_v[c // 8, pl.ds((c % 8) * _L, _L)] = j

        for g in range(bpw // _DP):
            r0 = base + g * _DP
            c0 = pltpu.async_copy(bout_hbm.at[pl.ds(r0, _DP)], bst_v, sem)
            c1 = pltpu.async_copy(aout_hbm.at[pl.ds(r0, _DP)], ast_v, sem1)
            c2 = pltpu.async_copy(ytab_hbm.at[ybkt_v.at[g]], yst_v, sem2)
            c0.wait()
            c1.wait()
            c2.wait()

            def row_body(r, _):
                rv = jnp.full((_L,), 0, jnp.int32) + r
                for t, buf in ((0, bst_v), (1, ast_v), (2, yst_v)):
                    for k in range(D // _L):
                        v = buf[r, pl.ds(k * _L, _L)]
                        plsc.store_scatter(
                            blk_v, [t * D + k * _L + lane, rv], v)
                return 0

            lax.fori_loop(0, _DP, row_body, 0)
            for q in range(_DP // _L):
                y = year_v[pl.ds(g * _DP + q * _L, _L)]
                ny = (y - mean) / std
                plsc.store_scatter(
                    blk_v,
                    [jnp.full((_L,), 3 * D, jnp.int32), q * _L + lane], ny)
            pltpu.sync_copy(blk_v, out_hbm.at[:, pl.ds(r0, _DP)])

    return gather_k, asm_k


def kernel(isbn_idx, author_idx, year_of_publication, book_table,
           author_table, year_table, boundaries, year_mean, year_std):
    B = isbn_idx.shape[0]
    D = book_table.shape[1]
    NB = boundaries.shape[0]
    NBOOK = book_table.shape[0]
    NAUTH = author_table.shape[0]
    gather_k, asm_k = _build(B, D, NB, NBOOK, NAUTH)

    def tail_of(tab):
        t0 = (tab.shape[0] // _DP) * _DP
        tl = tab[t0:]
        return jnp.pad(tl, ((0, _DP - tl.shape[0]), (0, 0))).T

    book_out, auth_out = gather_k(
        isbn_idx, author_idx, book_table.T, author_table.T,
        tail_of(book_table), tail_of(author_table))

    year128 = jnp.pad(year_table, ((0, 0), (0, _DP - D)))
    neg = jnp.full((1,), -jnp.inf, dtype=jnp.float32)
    pos = jnp.full((3,), jnp.inf, dtype=jnp.float32)
    bpad = jnp.concatenate([neg, boundaries.astype(jnp.float32), pos])
    consts = jnp.concatenate([
        jnp.full((_L,), year_mean, dtype=jnp.float32),
        jnp.full((_L,), year_std, dtype=jnp.float32),
    ])
    out_t = asm_k(year_of_publication, book_out, auth_out, year128,
                  bpad, consts)
    return out_t.T


# speedup vs baseline: 1.3153x; 1.0064x over previous
import functools

import jax
import jax.numpy as jnp
from jax import lax
from jax.experimental import pallas as pl
from jax.experimental.pallas import tpu as pltpu
from jax.experimental.pallas import tpu_sc as plsc

_L = 16
_DP = 128
_CW = 256
_SLOTS = 640
_SUP = 384
_BIG = 0x7FFFFFF


def _ceil_mult(x, m):
    return (x + m - 1) // m * m


@functools.lru_cache(maxsize=None)
def _build(B, D, NB, NBOOK, NAUTH):
    info = plsc.get_sparse_core_info()
    NC, NS = info.num_cores, info.num_subcores
    NW = NC * NS
    bpw = B // NW
    W = 3 * D + 1
    BD = B + _DP
    mesh = plsc.VectorSubcoreMesh(core_axis_name="c", subcore_axis_name="s")

    def geom(n):
        slab = _ceil_mult(_ceil_mult(n, NW) // NW, _DP)
        nsup = (slab + 8 * _CW - 1) // (8 * _CW)
        tail0 = (n // _DP) * _DP
        clamp = max(0, tail0 - _CW)
        return slab, nsup, tail0, clamp

    BSLAB, BNSUP, BTAIL0, BCLAMP = geom(NBOOK)
    ASLAB, ANSUP, ATAIL0, ACLAMP = geom(NAUTH)

    @functools.partial(
        pl.kernel,
        mesh=mesh,
        compiler_params=pltpu.CompilerParams(needs_layout_passes=False),
        out_type=(
            jax.ShapeDtypeStruct((BD, _DP), jnp.float32),
            jax.ShapeDtypeStruct((BD, _DP), jnp.float32),
        ),
        scratch_types=[
            pltpu.VMEM((2048,), jnp.int32),
            pltpu.VMEM((_DP, _CW), jnp.float32),
            pltpu.VMEM((_SLOTS, _DP), jnp.float32),
            pltpu.VMEM((_SLOTS,), jnp.int32),
            pltpu.VMEM((_SLOTS // _DP, _DP), jnp.int32),
            pltpu.VMEM((_DP,), jnp.int32),
            pltpu.VMEM((_DP,), jnp.int32),
            pltpu.VMEM((_SUP,), jnp.int32),
            pltpu.VMEM((_SUP,), jnp.int32),
            pltpu.SemaphoreType.DMA,
            pltpu.SemaphoreType.DMA,
        ],
    )
    def gather_k(isbn_hbm, auth_hbm, bt_hbm, at_hbm, btail_hbm, atail_hbm,
                 bout_hbm, aout_hbm,
                 idx_v, chunk_v, rows_v, sr_v, pidx_v, ccc_v, ccs_v,
                 sr2_v, ss2_v, sem0, sem1):
        wid = lax.axis_index("s") * NC + lax.axis_index("c")
        lane = jnp.arange(_L, dtype=jnp.int32)

        def run_table(idx_hbm, tab_hbm, tail_hbm, out_hbm,
                      slab, nsup, tail0, clamp):
            nch = nsup * 8
            lo = wid * slab
            hi = lo + slab
            for k in range(_SLOTS // _L):
                sr_v[pl.ds(k * _L, _L)] = jnp.full((_L,), _BIG, jnp.int32)
            for k in range(_SLOTS // _DP):
                for q in range(_DP // _L):
                    pidx_v[k, pl.ds(q * _L, _L)] = jnp.full(
                        (_L,), B + 7, jnp.int32)

            def pre_outer(p, cnt):
                pltpu.sync_copy(idx_hbm.at[pl.ds(p * 2048, 2048)], idx_v)

                def pre_body(g, cnt):
                    v = idx_v[pl.ds(g * _L, _L)]
                    m = (v >= lo) & (v < hi)
                    n = jnp.sum(jnp.where(m, 1, 0))

                    @pl.when(n > 0)
                    def _():
                        pos = cnt + plsc.cumsum(jnp.where(m, 1, 0)) - 1
                        pos = jnp.minimum(pos, _SLOTS - 1)
                        plsc.store_scatter(sr_v, [pos], v, mask=m)
                        plsc.store_scatter(
                            pidx_v, [pos >> 7, pos & 127],
                            p * 2048 + g * _L + lane, mask=m)

                    return cnt + n

                return lax.fori_loop(0, 2048 // _L, pre_body, cnt)

            cnt = lax.fori_loop(0, B // 2048, pre_outer, jnp.int32(0))
            ngrp = jnp.minimum((cnt + _L - 1) // _L, _SLOTS // _L)

            def extract_chunk(o, pair, width, bound, from_super):
                def cl_body(k, ccnt):
                    for u in range(2):
                        g = k * 2 + u
                        if from_super:
                            rv = sr2_v[pl.ds(g * _L, _L)]
                            sv = ss2_v[pl.ds(g * _L, _L)]
                        else:
                            rv = sr_v[pl.ds(g * _L, _L)]
                            sv = g * _L + lane
                        m = (rv >= o) & (rv < o + width)
                        n = jnp.sum(jnp.where(m, 1, 0))

                        @pl.when(n > 0)
                        def _(ccnt=ccnt, m=m, rv=rv, sv=sv):
                            pos = ccnt + plsc.cumsum(jnp.where(m, 1, 0)) - 1
                            pos = jnp.minimum(pos, _DP - 1)
                            plsc.store_scatter(ccc_v, [pos], rv - o, mask=m)
                            plsc.store_scatter(ccs_v, [pos], sv, mask=m)

                        ccnt = ccnt + n
                    return ccnt

                ccnt = lax.fori_loop(0, (bound + 1) // 2, cl_body, jnp.int32(0))
                ccnt = jnp.minimum(ccnt, _DP)

                def ex_body(e, _):
                    cc = ccc_v[pl.ds(e * _L, _L)]
                    slot = ccs_v[pl.ds(e * _L, _L)]
                    em = (e * _L + lane) < ccnt
                    base_r = pair * 64
                    for c in range(D):
                        rowv = base_r + jnp.full((_L,), c, jnp.int32)
                        val = plsc.load_gather(
                            chunk_v, [rowv, cc], mask=em)
                        plsc.store_scatter(
                            rows_v, [slot, jnp.full((_L,), c, jnp.int32)],
                            val, mask=em)
                    return 0

                lax.fori_loop(0, (ccnt + _L - 1) // _L, ex_body, 0)

            def off(j):
                return jnp.minimum(lo + j * _CW, clamp)

            def fire_dyn(j):
                def issue(sem):
                    o = off(j)
                    par = j & 1
                    for b in range(8):
                        pltpu.async_copy(
                            tab_hbm.at[pl.ds(b * 8, 8), pl.ds(o, _CW)],
                            chunk_v.at[pl.ds(par * 64 + b * 8, 8), :], sem)

                @pl.when((j & 1) == 0)
                def _():
                    issue(sem0)

                @pl.when((j & 1) == 1)
                def _():
                    issue(sem1)

            def drain_dyn(j):
                def dr(sem):
                    o = off(j)
                    par = j & 1
                    for b in range(8):
                        pltpu.make_async_copy(
                            tab_hbm.at[pl.ds(b * 8, 8), pl.ds(o, _CW)],
                            chunk_v.at[pl.ds(par * 64 + b * 8, 8), :],
                            sem).wait()

                @pl.when((j & 1) == 0)
                def _():
                    dr(sem0)

                @pl.when((j & 1) == 1)
                def _():
                    dr(sem1)

            fire_dyn(jnp.int32(0))

            def sup_body(s, _):
                j0 = s * 8
                lo_s = off(j0)
                hi_s = off(j0 + 7) + _CW
                for k in range(_SUP // _L):
                    sr2_v[pl.ds(k * _L, _L)] = jnp.full(
                        (_L,), _BIG, jnp.int32)

                def sl_body(k, scnt):
                    for u in range(2):
                        g = k * 2 + u
                        rv = sr_v[pl.ds(g * _L, _L)]
                        m = (rv >= lo_s) & (rv < hi_s)
                        n = jnp.sum(jnp.where(m, 1, 0))

                        @pl.when(n > 0)
                        def _(scnt=scnt, m=m, rv=rv, g=g):
                            pos = scnt + plsc.cumsum(jnp.where(m, 1, 0)) - 1
                            pos = jnp.minimum(pos, _SUP - 1)
                            plsc.store_scatter(sr2_v, [pos], rv, mask=m)
                            plsc.store_scatter(
                                ss2_v, [pos], g * _L + lane, mask=m)

                        scnt = scnt + n
                    return scnt

                scnt = lax.fori_loop(0, (ngrp + 1) // 2, sl_body,
                                     jnp.int32(0))
                sgrp = jnp.minimum((scnt + _L - 1) // _L, _SUP // _L)

                def ch_body(j2, _):
                    j = j0 + j2

                    @pl.when(j + 1 < nch)
                    def _():
                        fire_dyn(j + 1)

                    drain_dyn(j)
                    extract_chunk(off(j), j & 1, _CW, sgrp, True)
                    return 0

                lax.fori_loop(0, 8, ch_body, 0)
                return 0

            lax.fori_loop(0, nsup, sup_body, 0)

            pltpu.sync_copy(tail_hbm, chunk_v.at[pl.ds(0, 64), pl.ds(0, _DP)])
            extract_chunk(jnp.int32(tail0), jnp.int32(0), _DP, ngrp, False)

            cps = []
            for k in range(_SLOTS // _DP):
                cps.append(pltpu.async_copy(
                    rows_v.at[pl.ds(k * _DP, _DP)],
                    out_hbm.at[pidx_v.at[k]],
                    sem0))
            for cp in cps:
                cp.wait()

        run_table(isbn_hbm, bt_hbm, btail_hbm, bout_hbm,
                  BSLAB, BNSUP, BTAIL0, BCLAMP)
        run_table(auth_hbm, at_hbm, atail_hbm, aout_hbm,
                  ASLAB, ANSUP, ATAIL0, ACLAMP)

    @functools.partial(
        pl.kernel,
        mesh=mesh,
        compiler_params=pltpu.CompilerParams(needs_layout_passes=False),
        out_type=jax.ShapeDtypeStruct((W, B), jnp.float32),
        scratch_types=[
            pltpu.VMEM((bpw,), jnp.float32),
            pltpu.VMEM((bpw // _DP, _DP), jnp.int32),
            pltpu.VMEM((NB + 4,), jnp.float32),
            pltpu.VMEM((2 * _L,), jnp.float32),
            pltpu.VMEM((_DP, _DP), jnp.float32),
            pltpu.VMEM((_DP, _DP), jnp.float32),
            pltpu.VMEM((_DP, _DP), jnp.float32),
            pltpu.VMEM((W, _DP), jnp.float32),
            pltpu.SemaphoreType.DMA,
            pltpu.SemaphoreType.DMA,
            pltpu.SemaphoreType.DMA,
        ],
    )
    def asm_k(year_hbm, bout_hbm, aout_hbm, ytab_hbm, bpad_hbm, consts_hbm,
              out_hbm,
              year_v, ybkt_v, bpad_v, consts_v, bst_v, ast_v, yst_v, blk_v,
              sem, sem1, sem2):
        wid = lax.axis_index("s") * NC + lax.axis_index("c")
        lane = jnp.arange(_L, dtype=jnp.int32)
        base = wid * bpw
        pltpu.sync_copy(year_hbm.at[pl.ds(base, bpw)], year_v)
        pltpu.sync_copy(bpad_hbm, bpad_v)
        pltpu.sync_copy(consts_hbm, consts_v)

        mean = consts_v[pl.ds(0, _L)]
        std = consts_v[pl.ds(_L, _L)]
        scale = jnp.float32(NB - 1)
        for c in range(bpw // _L):
            y = year_v[pl.ds(c * _L, _L)]
            j = jnp.clip((y * scale).astype(jnp.int32) + 1, 0, NB)
            for _ in range(2):
                hi = plsc.load_gather(bpad_v, [j + 1])
                lo2 = plsc.load_gather(bpad_v, [j])
                j = j + jnp.where(hi <= y, 1, 0) - jnp.where(lo2 > y, 1, 0)
            ybkt_v[c // 8, pl.ds((c % 8) * _L, _L)] = j

        for g in range(bpw // _DP):
            r0 = base + g * _DP
            c0 = pltpu.async_copy(bout_hbm.at[pl.ds(r0, _DP)], bst_v, sem)
            c1 = pltpu.async_copy(aout_hbm.at[pl.ds(r0, _DP)], ast_v, sem1)
            c2 = pltpu.async_copy(ytab_hbm.at[ybkt_v.at[g]], yst_v, sem2)
            c0.wait()
            c1.wait()
            c2.wait()

            def row_body(r, _):
                rv = jnp.full((_L,), 0, jnp.int32) + r
                for t, buf in ((0, bst_v), (1, ast_v), (2, yst_v)):
                    for k in range(D // _L):
                        v = buf[r, pl.ds(k * _L, _L)]
                        plsc.store_scatter(
                            blk_v, [t * D + k * _L + lane, rv], v)
                return 0

            lax.fori_loop(0, _DP, row_body, 0)
            for q in range(_DP // _L):
                y = year_v[pl.ds(g * _DP + q * _L, _L)]
                ny = (y - mean) / std
                plsc.store_scatter(
                    blk_v,
                    [jnp.full((_L,), 3 * D, jnp.int32), q * _L + lane], ny)
            pltpu.sync_copy(blk_v, out_hbm.at[:, pl.ds(r0, _DP)])

    return gather_k, asm_k


def kernel(isbn_idx, author_idx, year_of_publication, book_table,
           author_table, year_table, boundaries, year_mean, year_std):
    B = isbn_idx.shape[0]
    D = book_table.shape[1]
    NB = boundaries.shape[0]
    NBOOK = book_table.shape[0]
    NAUTH = author_table.shape[0]
    gather_k, asm_k = _build(B, D, NB, NBOOK, NAUTH)

    def tail_of(tab):
        t0 = (tab.shape[0] // _DP) * _DP
        tl = tab[t0:]
        return jnp.pad(tl, ((0, _DP - tl.shape[0]), (0, 0))).T

    book_out, auth_out = gather_k(
        isbn_idx, author_idx, book_table.T, author_table.T,
        tail_of(book_table), tail_of(author_table))

    year128 = jnp.pad(year_table, ((0, 0), (0, _DP - D)))
    neg = jnp.full((1,), -jnp.inf, dtype=jnp.float32)
    pos = jnp.full((3,), jnp.inf, dtype=jnp.float32)
    bpad = jnp.concatenate([neg, boundaries.astype(jnp.float32), pos])
    consts = jnp.concatenate([
        jnp.full((_L,), year_mean, dtype=jnp.float32),
        jnp.full((_L,), year_std, dtype=jnp.float32),
    ])
    out_t = asm_k(year_of_publication, book_out, auth_out, year128,
                  bpad, consts)
    return out_t.T


# speedup vs baseline: 1.3193x; 1.0030x over previous
import functools

import jax
import jax.numpy as jnp
from jax import lax
from jax.experimental import pallas as pl
from jax.experimental.pallas import tpu as pltpu
from jax.experimental.pallas import tpu_sc as plsc

_L = 16
_DP = 128
_CW = 256
_SLOTS = 640
_SUP = 384
_BIG = 0x7FFFFFF


def _ceil_mult(x, m):
    return (x + m - 1) // m * m


@functools.lru_cache(maxsize=None)
def _build(B, D, NB, NBOOK, NAUTH):
    info = plsc.get_sparse_core_info()
    NC, NS = info.num_cores, info.num_subcores
    NW = NC * NS
    bpw = B // NW
    W = 3 * D + 1
    BD = B + _DP
    mesh = plsc.VectorSubcoreMesh(core_axis_name="c", subcore_axis_name="s")

    def geom(n):
        slab = _ceil_mult(_ceil_mult(n, NW) // NW, _DP)
        nsup = (slab + 8 * _CW - 1) // (8 * _CW)
        tail0 = (n // _DP) * _DP
        clamp = max(0, tail0 - _CW)
        return slab, nsup, tail0, clamp

    BSLAB, BNSUP, BTAIL0, BCLAMP = geom(NBOOK)
    ASLAB, ANSUP, ATAIL0, ACLAMP = geom(NAUTH)

    @functools.partial(
        pl.kernel,
        mesh=mesh,
        compiler_params=pltpu.CompilerParams(needs_layout_passes=False),
        out_type=(
            jax.ShapeDtypeStruct((BD, _DP), jnp.float32),
            jax.ShapeDtypeStruct((BD, _DP), jnp.float32),
        ),
        scratch_types=[
            pltpu.VMEM((2048,), jnp.int32),
            pltpu.VMEM((_DP, _CW), jnp.float32),
            pltpu.VMEM((_SLOTS, _DP), jnp.float32),
            pltpu.VMEM((_SLOTS,), jnp.int32),
            pltpu.VMEM((_SLOTS // _DP, _DP), jnp.int32),
            pltpu.VMEM((_DP,), jnp.int32),
            pltpu.VMEM((_DP,), jnp.int32),
            pltpu.VMEM((_SUP,), jnp.int32),
            pltpu.VMEM((_SUP,), jnp.int32),
            pltpu.SemaphoreType.DMA,
            pltpu.SemaphoreType.DMA,
        ],
    )
    def gather_k(isbn_hbm, auth_hbm, bt_hbm, at_hbm, btail_hbm, atail_hbm,
                 bout_hbm, aout_hbm,
                 idx_v, chunk_v, rows_v, sr_v, pidx_v, ccc_v, ccs_v,
                 sr2_v, ss2_v, sem0, sem1):
        wid = lax.axis_index("s") * NC + lax.axis_index("c")
        lane = jnp.arange(_L, dtype=jnp.int32)

        def run_table(idx_hbm, tab_hbm, tail_hbm, out_hbm,
                      slab, nsup, tail0, clamp):
            nch = nsup * 8
            lo = wid * slab
            hi = lo + slab
            for k in range(_SLOTS // _L):
                sr_v[pl.ds(k * _L, _L)] = jnp.full((_L,), _BIG, jnp.int32)
            for k in range(_SLOTS // _DP):
                for q in range(_DP // _L):
                    pidx_v[k, pl.ds(q * _L, _L)] = jnp.full(
                        (_L,), B + 7, jnp.int32)

            def pre_outer(p, cnt):
                pltpu.sync_copy(idx_hbm.at[pl.ds(p * 2048, 2048)], idx_v)

                def pre_body(g, cnt):
                    v = idx_v[pl.ds(g * _L, _L)]
                    m = (v >= lo) & (v < hi)
                    n = jnp.sum(jnp.where(m, 1, 0))

                    @pl.when(n > 0)
                    def _():
                        pos = cnt + plsc.cumsum(jnp.where(m, 1, 0)) - 1
                        pos = jnp.minimum(pos, _SLOTS - 1)
                        plsc.store_scatter(sr_v, [pos], v, mask=m)
                        plsc.store_scatter(
                            pidx_v, [pos >> 7, pos & 127],
                            p * 2048 + g * _L + lane, mask=m)

                    return cnt + n

                return lax.fori_loop(0, 2048 // _L, pre_body, cnt)

            cnt = lax.fori_loop(0, B // 2048, pre_outer, jnp.int32(0))
            ngrp = jnp.minimum((cnt + _L - 1) // _L, _SLOTS // _L)

            def extract_chunk(o, pair, width, bound, from_super):
                def cl_body(k, ccnt):
                    for u in range(2):
                        g = k * 2 + u
                        if from_super:
                            rv = sr2_v[pl.ds(g * _L, _L)]
                            sv = ss2_v[pl.ds(g * _L, _L)]
                        else:
                            rv = sr_v[pl.ds(g * _L, _L)]
                            sv = g * _L + lane
                        m = (rv >= o) & (rv < o + width)
                        n = jnp.sum(jnp.where(m, 1, 0))

                        @pl.when(n > 0)
                        def _(ccnt=ccnt, m=m, rv=rv, sv=sv):
                            pos = ccnt + plsc.cumsum(jnp.where(m, 1, 0)) - 1
                            pos = jnp.minimum(pos, _DP - 1)
                            plsc.store_scatter(ccc_v, [pos], rv - o, mask=m)
                            plsc.store_scatter(ccs_v, [pos], sv, mask=m)

                        ccnt = ccnt + n
                    return ccnt

                ccnt = lax.fori_loop(0, (bound + 1) // 2, cl_body, jnp.int32(0))
                ccnt = jnp.minimum(ccnt, _DP)

                def ex_body(e, _):
                    cc = ccc_v[pl.ds(e * _L, _L)]
                    slot = ccs_v[pl.ds(e * _L, _L)]
                    em = (e * _L + lane) < ccnt
                    base_r = pair * 64
                    for c in range(D):
                        rowv = base_r + jnp.full((_L,), c, jnp.int32)
                        val = plsc.load_gather(
                            chunk_v, [rowv, cc], mask=em)
                        plsc.store_scatter(
                            rows_v, [slot, jnp.full((_L,), c, jnp.int32)],
                            val, mask=em)
                    return 0

                lax.fori_loop(0, (ccnt + _L - 1) // _L, ex_body, 0)

            def off(j):
                return jnp.minimum(lo + j * _CW, clamp)

            def fire_dyn(j):
                def issue(sem):
                    o = off(j)
                    par = j & 1
                    for b in range(8):
                        pltpu.async_copy(
                            tab_hbm.at[pl.ds(b * 8, 8), pl.ds(o, _CW)],
                            chunk_v.at[pl.ds(par * 64 + b * 8, 8), :], sem)

                @pl.when((j & 1) == 0)
                def _():
                    issue(sem0)

                @pl.when((j & 1) == 1)
                def _():
                    issue(sem1)

            def drain_dyn(j):
                def dr(sem):
                    o = off(j)
                    par = j & 1
                    for b in range(8):
                        pltpu.make_async_copy(
                            tab_hbm.at[pl.ds(b * 8, 8), pl.ds(o, _CW)],
                            chunk_v.at[pl.ds(par * 64 + b * 8, 8), :],
                            sem).wait()

                @pl.when((j & 1) == 0)
                def _():
                    dr(sem0)

                @pl.when((j & 1) == 1)
                def _():
                    dr(sem1)

            fire_dyn(jnp.int32(0))

            def sup_body(s, _):
                j0 = s * 8
                lo_s = off(j0)
                hi_s = off(j0 + 7) + _CW
                for k in range(_SUP // _L):
                    sr2_v[pl.ds(k * _L, _L)] = jnp.full(
                        (_L,), _BIG, jnp.int32)

                def sl_body(k, scnt):
                    for u in range(2):
                        g = k * 2 + u
                        rv = sr_v[pl.ds(g * _L, _L)]
                        m = (rv >= lo_s) & (rv < hi_s)
                        n = jnp.sum(jnp.where(m, 1, 0))

                        @pl.when(n > 0)
                        def _(scnt=scnt, m=m, rv=rv, g=g):
                            pos = scnt + plsc.cumsum(jnp.where(m, 1, 0)) - 1
                            pos = jnp.minimum(pos, _SUP - 1)
                            plsc.store_scatter(sr2_v, [pos], rv, mask=m)
                            plsc.store_scatter(
                                ss2_v, [pos], g * _L + lane, mask=m)

                        scnt = scnt + n
                    return scnt

                scnt = lax.fori_loop(0, (ngrp + 1) // 2, sl_body,
                                     jnp.int32(0))
                sgrp = jnp.minimum((scnt + _L - 1) // _L, _SUP // _L)

                def ch_body(j2, _):
                    j = j0 + j2

                    @pl.when(j + 1 < nch)
                    def _():
                        fire_dyn(j + 1)

                    drain_dyn(j)
                    extract_chunk(off(j), j & 1, _CW, sgrp, True)
                    return 0

                lax.fori_loop(0, 8, ch_body, 0)
                return 0

            lax.fori_loop(0, nsup, sup_body, 0)

            pltpu.sync_copy(tail_hbm, chunk_v.at[pl.ds(0, 64), pl.ds(0, _DP)])
            extract_chunk(jnp.int32(tail0), jnp.int32(0), _DP, ngrp, False)

            cps = []
            for k in range(_SLOTS // _DP):
                cps.append(pltpu.async_copy(
                    rows_v.at[pl.ds(k * _DP, _DP)],
                    out_hbm.at[pidx_v.at[k]],
                    sem0))
            for cp in cps:
                cp.wait()

        run_table(isbn_hbm, bt_hbm, btail_hbm, bout_hbm,
                  BSLAB, BNSUP, BTAIL0, BCLAMP)
        run_table(auth_hbm, at_hbm, atail_hbm, aout_hbm,
                  ASLAB, ANSUP, ATAIL0, ACLAMP)

    @functools.partial(
        pl.kernel,
        mesh=mesh,
        compiler_params=pltpu.CompilerParams(needs_layout_passes=False),
        out_type=jax.ShapeDtypeStruct((W, B), jnp.float32),
        scratch_types=[
            pltpu.VMEM((bpw,), jnp.float32),
            pltpu.VMEM((bpw // _DP, _DP), jnp.int32),
            pltpu.VMEM((NB + 4,), jnp.float32),
            pltpu.VMEM((2 * _L,), jnp.float32),
            pltpu.VMEM((_DP, _DP), jnp.float32),
            pltpu.VMEM((_DP, _DP), jnp.float32),
            pltpu.VMEM((_DP, _DP), jnp.float32),
            pltpu.VMEM((W, _DP), jnp.float32),
            pltpu.VMEM((W, _DP), jnp.float32),
            pltpu.SemaphoreType.DMA,
            pltpu.SemaphoreType.DMA,
            pltpu.SemaphoreType.DMA,
            pltpu.SemaphoreType.DMA,
            pltpu.SemaphoreType.DMA,
        ],
    )
    def asm_k(year_hbm, bout_hbm, aout_hbm, ytab_hbm, bpad_hbm, consts_hbm,
              out_hbm,
              year_v, ybkt_v, bpad_v, consts_v, bst_v, ast_v, yst_v,
              blk0_v, blk1_v, sem, sem1, sem2, sem3, sem4):
        wid = lax.axis_index("s") * NC + lax.axis_index("c")
        lane = jnp.arange(_L, dtype=jnp.int32)
        base = wid * bpw
        pltpu.sync_copy(year_hbm.at[pl.ds(base, bpw)], year_v)
        pltpu.sync_copy(bpad_hbm, bpad_v)
        pltpu.sync_copy(consts_hbm, consts_v)

        mean = consts_v[pl.ds(0, _L)]
        std = consts_v[pl.ds(_L, _L)]
        scale = jnp.float32(NB - 1)
        for c in range(bpw // _L):
            y = year_v[pl.ds(c * _L, _L)]
            j = jnp.clip((y * scale).astype(jnp.int32) + 1, 0, NB)
            for _ in range(2):
                hi = plsc.load_gather(bpad_v, [j + 1])
                lo2 = plsc.load_gather(bpad_v, [j])
                j = j + jnp.where(hi <= y, 1, 0) - jnp.where(lo2 > y, 1, 0)
            ybkt_v[c // 8, pl.ds((c % 8) * _L, _L)] = j

        blks = (blk0_v, blk1_v)
        outcps = [None, None]
        for g in range(bpw // _DP):
            r0 = base + g * _DP
            blk_v = blks[g % 2]
            c0 = pltpu.async_copy(bout_hbm.at[pl.ds(r0, _DP)], bst_v, sem)
            c1 = pltpu.async_copy(aout_hbm.at[pl.ds(r0, _DP)], ast_v, sem1)
            c2 = pltpu.async_copy(ytab_hbm.at[ybkt_v.at[g]], yst_v, sem2)
            if outcps[g % 2] is not None:
                outcps[g % 2].wait()
            c0.wait()
            c1.wait()
            c2.wait()

            def row_body(r, _, blk_v=blk_v):
                rv = jnp.full((_L,), 0, jnp.int32) + r
                for rr in range(2):
                    for t, buf in ((0, bst_v), (1, ast_v), (2, yst_v)):
                        for k in range(D // _L):
                            v = buf[r * 2 + rr, pl.ds(k * _L, _L)]
                            plsc.store_scatter(
                                blk_v, [t * D + k * _L + lane, rv * 2 + rr],
                                v)
                return 0

            lax.fori_loop(0, _DP // 2, row_body, 0)
            for q in range(_DP // _L):
                y = year_v[pl.ds(g * _DP + q * _L, _L)]
                ny = (y - mean) / std
                plsc.store_scatter(
                    blk_v,
                    [jnp.full((_L,), 3 * D, jnp.int32), q * _L + lane], ny)
            outcps[g % 2] = pltpu.async_copy(
                blk_v, out_hbm.at[:, pl.ds(r0, _DP)],
                sem3 if g % 2 == 0 else sem4)
        for cp in outcps:
            if cp is not None:
                cp.wait()

    return gather_k, asm_k


def kernel(isbn_idx, author_idx, year_of_publication, book_table,
           author_table, year_table, boundaries, year_mean, year_std):
    B = isbn_idx.shape[0]
    D = book_table.shape[1]
    NB = boundaries.shape[0]
    NBOOK = book_table.shape[0]
    NAUTH = author_table.shape[0]
    gather_k, asm_k = _build(B, D, NB, NBOOK, NAUTH)

    def tail_of(tab):
        t0 = (tab.shape[0] // _DP) * _DP
        tl = tab[t0:]
        return jnp.pad(tl, ((0, _DP - tl.shape[0]), (0, 0))).T

    book_out, auth_out = gather_k(
        isbn_idx, author_idx, book_table.T, author_table.T,
        tail_of(book_table), tail_of(author_table))

    year128 = jnp.pad(year_table, ((0, 0), (0, _DP - D)))
    neg = jnp.full((1,), -jnp.inf, dtype=jnp.float32)
    pos = jnp.full((3,), jnp.inf, dtype=jnp.float32)
    bpad = jnp.concatenate([neg, boundaries.astype(jnp.float32), pos])
    consts = jnp.concatenate([
        jnp.full((_L,), year_mean, dtype=jnp.float32),
        jnp.full((_L,), year_std, dtype=jnp.float32),
    ])
    out_t = asm_k(year_of_publication, book_out, auth_out, year128,
                  bpad, consts)
    return out_t.T
